# baseline (speedup 1.0000x reference)
import functools

import jax
import jax.numpy as jnp
from jax import lax
from jax.experimental import pallas as pl
from jax.experimental.pallas import tpu as pltpu
from jax.experimental.pallas import tpu_sc as plsc

N = 10000
E = 160000
D_IN = 256
D_HID = 256
D_OUT = 128

NC = 2
NS = 16
LANES = 16

N_PAD = 10240
E_PAD = 163840
EB = 128
NB_ALL = E_PAD // EB
ROW_BLK = 512
GRID_R = N_PAD // ROW_BLK

_MESH = plsc.VectorSubcoreMesh(
    core_axis_name="c", subcore_axis_name="s", num_cores=NC, num_subcores=NS
)


def _zero_rows(rows, nrow, ncol):
    zero16 = jnp.zeros((LANES,), jnp.float32)

    def _zr(i, carry):
        def _zc(j, carry2):
            rows[i, pl.ds(j * LANES, LANES)] = zero16
            return carry2

        return lax.fori_loop(0, ncol // LANES, _zc, carry)

    lax.fori_loop(0, nrow, _zr, 0)


CHUNK = 40


def _edge_pipeline(t_hbm, src2_hbm, dst2_hbm, batch0, nb,
                   sidx2, didx2, dbuf, rows0, rows1, sem0, sem1, acc):
    def _scatter(b, rows):
        for j in range(EB // LANES):
            dbuf[pl.ds(j * LANES, LANES)] = didx2[b, pl.ds(j * LANES, LANES)]
        pltpu.sync_copy(rows, acc.at[dbuf], add=True)

    def _chunk(ci, carry):
        boff = batch0 + ci * CHUNK
        pltpu.sync_copy(src2_hbm.at[pl.ds(boff, CHUNK)], sidx2)
        pltpu.sync_copy(dst2_hbm.at[pl.ds(boff, CHUNK)], didx2)
        pltpu.async_copy(t_hbm.at[sidx2.at[0]], rows0, sem0)

        def _body(b2, carry2):
            b = b2 * 2
            pltpu.async_copy(t_hbm.at[sidx2.at[b + 1]], rows1, sem1)
            pltpu.make_async_copy(t_hbm.at[sidx2.at[b]], rows0, sem0).wait()
            _scatter(b, rows0)
            pltpu.async_copy(t_hbm.at[sidx2.at[b + 2]], rows0, sem0)
            pltpu.make_async_copy(
                t_hbm.at[sidx2.at[b + 1]], rows1, sem1).wait()
            _scatter(b + 1, rows1)
            return carry2

        lax.fori_loop(0, CHUNK // 2 - 1, _body, 0)
        b = CHUNK - 2
        pltpu.async_copy(t_hbm.at[sidx2.at[b + 1]], rows1, sem1)
        pltpu.make_async_copy(t_hbm.at[sidx2.at[b]], rows0, sem0).wait()
        _scatter(b, rows0)
        pltpu.make_async_copy(t_hbm.at[sidx2.at[b + 1]], rows1, sem1).wait()
        _scatter(b + 1, rows1)
        return carry

    lax.fori_loop(0, nb // CHUNK, _chunk, 0)


@functools.partial(
    pl.kernel,
    out_type=jax.ShapeDtypeStruct((NC, N_PAD), jnp.float32),
    mesh=_MESH,
    scratch_types=[
        pltpu.VMEM((NB_ALL // (NC * NS), EB), jnp.int32),
        pltpu.VMEM((EB,), jnp.int32),
        pltpu.VMEM((EB,), jnp.float32),
        pltpu.VMEM((N_PAD // NS,), jnp.float32),
        pltpu.VMEM_SHARED((N_PAD,), jnp.float32),
        pltpu.SemaphoreType.DMA,
    ],
)
def _deg_sc(dst2_hbm, cnt_hbm, didx2, dbuf, ones_v, bounce, acc, sem):
    c = lax.axis_index("c")
    s = lax.axis_index("s")
    rpt = N_PAD // NS
    nb = NB_ALL // (NC * NS)

    one16 = jnp.ones((LANES,), jnp.float32)
    zero16 = jnp.zeros((LANES,), jnp.float32)
    for j in range(EB // LANES):
        ones_v[pl.ds(j * LANES, LANES)] = one16

    def _zb(i, carry):
        bounce[pl.ds(i * LANES, LANES)] = zero16
        return carry

    lax.fori_loop(0, rpt // LANES, _zb, 0)
    w = c * NS + s
    pltpu.sync_copy(dst2_hbm.at[pl.ds(w * nb, nb)], didx2)
    pltpu.sync_copy(bounce, acc.at[pl.ds(s * rpt, rpt)])
    plsc.subcore_barrier()

    def _body(b, carry):
        for j in range(EB // LANES):
            dbuf[pl.ds(j * LANES, LANES)] = didx2[b, pl.ds(j * LANES, LANES)]
        pltpu.sync_copy(ones_v, acc.at[dbuf], add=True)
        return carry

    lax.fori_loop(0, nb, _body, 0)
    plsc.subcore_barrier()

    pltpu.sync_copy(acc.at[pl.ds(s * rpt, rpt)], bounce)
    pltpu.sync_copy(bounce, cnt_hbm.at[c, pl.ds(s * rpt, rpt)])


_DH1 = D_HID // 2
_NB1 = E_PAD // (NS * EB)


@functools.partial(
    pl.kernel,
    out_type=(
        jax.ShapeDtypeStruct((N_PAD, _DH1), jnp.float32),
        jax.ShapeDtypeStruct((N_PAD, _DH1), jnp.float32),
    ),
    mesh=_MESH,
    scratch_types=[
        pltpu.VMEM((CHUNK, EB), jnp.int32),
        pltpu.VMEM((CHUNK, EB), jnp.int32),
        pltpu.VMEM((EB,), jnp.int32),
        pltpu.VMEM((EB, _DH1), jnp.float32),
        pltpu.VMEM((EB, _DH1), jnp.float32),
        pltpu.VMEM_SHARED((N_PAD, _DH1), jnp.float32),
        pltpu.SemaphoreType.DMA,
        pltpu.SemaphoreType.DMA,
    ],
)
def _prop_l1(src2_hbm, dst2_hbm, ta_hbm, tb_hbm, outa_hbm, outb_hbm,
             sidx2, didx2, dbuf, rows0, rows1, acc, sem0, sem1):
    c = lax.axis_index("c")
    s = lax.axis_index("s")
    rpt = N_PAD // NS

    _zero_rows(rows0, EB, _DH1)
    base = s * rpt
    for k in range(rpt // EB):
        pltpu.sync_copy(rows0, acc.at[pl.ds(base + k * EB, EB)])
    plsc.subcore_barrier()

    @pl.when(c == 0)
    def _():
        _edge_pipeline(ta_hbm, src2_hbm, dst2_hbm, s * _NB1, _NB1,
                       sidx2, didx2, dbuf, rows0, rows1, sem0, sem1, acc)

    @pl.when(c == 1)
    def _():
        _edge_pipeline(tb_hbm, src2_hbm, dst2_hbm, s * _NB1, _NB1,
                       sidx2, didx2, dbuf, rows0, rows1, sem0, sem1, acc)

    plsc.subcore_barrier()

    def _drain(out_hbm):
        pltpu.sync_copy(acc.at[pl.ds(base, rpt)], out_hbm.at[pl.ds(base, rpt)])

    @pl.when(c == 0)
    def _():
        _drain(outa_hbm)

    @pl.when(c == 1)
    def _():
        _drain(outb_hbm)


_NB2 = E_PAD // (NC * NS * EB)


@functools.partial(
    pl.kernel,
    out_type=(
        jax.ShapeDtypeStruct((N_PAD, D_OUT), jnp.float32),
        jax.ShapeDtypeStruct((N_PAD, D_OUT), jnp.float32),
    ),
    mesh=_MESH,
    scratch_types=[
        pltpu.VMEM((CHUNK, EB), jnp.int32),
        pltpu.VMEM((CHUNK, EB), jnp.int32),
        pltpu.VMEM((EB,), jnp.int32),
        pltpu.VMEM((EB, D_OUT), jnp.float32),
        pltpu.VMEM((EB, D_OUT), jnp.float32),
        pltpu.VMEM_SHARED((N_PAD, D_OUT), jnp.float32),
        pltpu.SemaphoreType.DMA,
        pltpu.SemaphoreType.DMA,
    ],
)
def _prop_l2(src2_hbm, dst2_hbm, t_hbm, outa_hbm, outb_hbm,
             sidx2, didx2, dbuf, rows0, rows1, acc, sem0, sem1):
    c = lax.axis_index("c")
    s = lax.axis_index("s")
    rpt = N_PAD // NS

    _zero_rows(rows0, EB, D_OUT)
    w = c * NS + s
    base = s * rpt
    for k in range(rpt // EB):
        pltpu.sync_copy(rows0, acc.at[pl.ds(base + k * EB, EB)])
    plsc.subcore_barrier()

    _edge_pipeline(t_hbm, src2_hbm, dst2_hbm, w * _NB2, _NB2,
                   sidx2, didx2, dbuf, rows0, rows1, sem0, sem1, acc)

    plsc.subcore_barrier()

    def _drain(out_hbm):
        pltpu.sync_copy(acc.at[pl.ds(base, rpt)], out_hbm.at[pl.ds(base, rpt)])

    @pl.when(c == 0)
    def _():
        _drain(outa_hbm)

    @pl.when(c == 1)
    def _():
        _drain(outb_hbm)


def _mm1_body(x_ref, w_ref, cnt_ref, ta_ref, tb_ref, dinv_ref):
    cnt = cnt_ref[0, :] + cnt_ref[1, :]
    dinv = lax.rsqrt(1.0 + cnt)
    h = jnp.dot(x_ref[...], w_ref[...], preferred_element_type=jnp.float32)
    h = h * dinv[:, None]
    ta_ref[...] = h[:, : D_HID // 2]
    tb_ref[...] = h[:, D_HID // 2 :]
    dinv_ref[...] = dinv


def _mm1(x_p, w1, cnt):
    return pl.pallas_call(
        _mm1_body,
        grid=(GRID_R,),
        in_specs=[
            pl.BlockSpec((ROW_BLK, D_IN), lambda i: (i, 0)),
            pl.BlockSpec((D_IN, D_HID), lambda i: (0, 0)),
            pl.BlockSpec((NC, ROW_BLK), lambda i: (0, i)),
        ],
        out_specs=[
            pl.BlockSpec((ROW_BLK, D_HID // 2), lambda i: (i, 0)),
            pl.BlockSpec((ROW_BLK, D_HID // 2), lambda i: (i, 0)),
            pl.BlockSpec((ROW_BLK,), lambda i: (i,)),
        ],
        out_shape=[
            jax.ShapeDtypeStruct((N_PAD, D_HID // 2), jnp.float32),
            jax.ShapeDtypeStruct((N_PAD, D_HID // 2), jnp.float32),
            jax.ShapeDtypeStruct((N_PAD,), jnp.float32),
        ],
    )(x_p, w1, cnt)


def _mm2_body(aa_ref, ab_ref, ta_ref, tb_ref, dinv_ref, b1_ref, w2_ref,
              t2_ref):
    dinv = dinv_ref[...]
    h1 = jnp.concatenate(
        [aa_ref[...] + ta_ref[...], ab_ref[...] + tb_ref[...]], axis=1
    )
    out1 = jnp.maximum(h1 * dinv[:, None] + b1_ref[...][None, :], 0.0)
    h2 = jnp.dot(out1, w2_ref[...], preferred_element_type=jnp.float32)
    t2_ref[...] = h2 * dinv[:, None]


def _mm2(aa, ab, ta, tb, dinv, b1, w2):
    return pl.pallas_call(
        _mm2_body,
        grid=(GRID_R,),
        in_specs=[
            pl.BlockSpec((ROW_BLK, D_HID // 2), lambda i: (i, 0)),
            pl.BlockSpec((ROW_BLK, D_HID // 2), lambda i: (i, 0)),
            pl.BlockSpec((ROW_BLK, D_HID // 2), lambda i: (i, 0)),
            pl.BlockSpec((ROW_BLK, D_HID // 2), lambda i: (i, 0)),
            pl.BlockSpec((ROW_BLK,), lambda i: (i,)),
            pl.BlockSpec((D_HID,), lambda i: (0,)),
            pl.BlockSpec((D_HID, D_OUT), lambda i: (0, 0)),
        ],
        out_specs=pl.BlockSpec((ROW_BLK, D_OUT), lambda i: (i, 0)),
        out_shape=jax.ShapeDtypeStruct((N_PAD, D_OUT), jnp.float32),
    )(aa, ab, ta, tb, dinv, b1, w2)


def _final_body(aa_ref, ab_ref, t2_ref, dinv_ref, b2_ref, out_ref):
    dinv = dinv_ref[...]
    h = aa_ref[...] + ab_ref[...] + t2_ref[...]
    out_ref[...] = h * dinv[:, None] + b2_ref[...][None, :]


def _final(aa, ab, t2, dinv, b2):
    return pl.pallas_call(
        _final_body,
        grid=(GRID_R,),
        in_specs=[
            pl.BlockSpec((ROW_BLK, D_OUT), lambda i: (i, 0)),
            pl.BlockSpec((ROW_BLK, D_OUT), lambda i: (i, 0)),
            pl.BlockSpec((ROW_BLK, D_OUT), lambda i: (i, 0)),
            pl.BlockSpec((ROW_BLK,), lambda i: (i,)),
            pl.BlockSpec((D_OUT,), lambda i: (0,)),
        ],
        out_specs=pl.BlockSpec((ROW_BLK, D_OUT), lambda i: (i, 0)),
        out_shape=jax.ShapeDtypeStruct((N_PAD, D_OUT), jnp.float32),
    )(aa, ab, t2, dinv, b2)


def kernel(x, edge_index, W1, b1, W2, b2):
    src = edge_index[0]
    dst = edge_index[1]
    pad_e = E_PAD - E
    pad_i = jnp.arange(pad_e, dtype=jnp.int32)
    src_p = jnp.concatenate([src, pad_i % N]).reshape(NB_ALL, EB)
    dst_p = jnp.concatenate(
        [dst, N + pad_i % (N_PAD - N)]
    ).reshape(NB_ALL, EB)
    x_p = jnp.pad(x, ((0, N_PAD - N), (0, 0)))

    cnt = _deg_sc(dst_p)
    ta, tb, dinv = _mm1(x_p, W1, cnt)
    aa, ab = _prop_l1(src_p, dst_p, ta, tb)
    t2 = _mm2(aa, ab, ta, tb, dinv, b1, W2)
    a2a, a2b = _prop_l2(src_p, dst_p, t2)
    out = _final(a2a, a2b, t2, dinv, b2)
    return out[:N]


# speedup vs baseline: 1.0252x; 1.0252x over previous
import functools

import jax
import jax.numpy as jnp
from jax import lax
from jax.experimental import pallas as pl
from jax.experimental.pallas import tpu as pltpu
from jax.experimental.pallas import tpu_sc as plsc

N = 10000
E = 160000
D_IN = 256
D_HID = 256
D_OUT = 128

NC = 2
NS = 16
LANES = 16

N_PAD = 10240
E_PAD = 163840
EB = 128
NB_ALL = E_PAD // EB
ROW_BLK = 512
GRID_R = N_PAD // ROW_BLK

_MESH = plsc.VectorSubcoreMesh(
    core_axis_name="c", subcore_axis_name="s", num_cores=NC, num_subcores=NS
)


def _zero_rows(rows, nrow, ncol):
    zero16 = jnp.zeros((LANES,), jnp.float32)

    def _zr(i, carry):
        def _zc(j, carry2):
            rows[i, pl.ds(j * LANES, LANES)] = zero16
            return carry2

        return lax.fori_loop(0, ncol // LANES, _zc, carry)

    lax.fori_loop(0, nrow, _zr, 0)


CHUNK = 40


def _edge_pipeline(t_hbm, src2_hbm, dst2_hbm, batch0, nb,
                   sidx2, didx2, dbuf, rows0, rows1, sem0, sem1, acc):
    def _scatter(b, rows):
        for j in range(EB // LANES):
            dbuf[pl.ds(j * LANES, LANES)] = didx2[b, pl.ds(j * LANES, LANES)]
        pltpu.sync_copy(rows, acc.at[dbuf], add=True)

    def _chunk(ci, carry):
        boff = batch0 + ci * CHUNK
        pltpu.sync_copy(src2_hbm.at[pl.ds(boff, CHUNK)], sidx2)
        pltpu.sync_copy(dst2_hbm.at[pl.ds(boff, CHUNK)], didx2)
        pltpu.async_copy(t_hbm.at[sidx2.at[0]], rows0, sem0)

        def _body(b2, carry2):
            b = b2 * 2
            pltpu.async_copy(t_hbm.at[sidx2.at[b + 1]], rows1, sem1)
            pltpu.make_async_copy(t_hbm.at[sidx2.at[b]], rows0, sem0).wait()
            _scatter(b, rows0)
            pltpu.async_copy(t_hbm.at[sidx2.at[b + 2]], rows0, sem0)
            pltpu.make_async_copy(
                t_hbm.at[sidx2.at[b + 1]], rows1, sem1).wait()
            _scatter(b + 1, rows1)
            return carry2

        lax.fori_loop(0, CHUNK // 2 - 1, _body, 0)
        b = CHUNK - 2
        pltpu.async_copy(t_hbm.at[sidx2.at[b + 1]], rows1, sem1)
        pltpu.make_async_copy(t_hbm.at[sidx2.at[b]], rows0, sem0).wait()
        _scatter(b, rows0)
        pltpu.make_async_copy(t_hbm.at[sidx2.at[b + 1]], rows1, sem1).wait()
        _scatter(b + 1, rows1)
        return carry

    lax.fori_loop(0, nb // CHUNK, _chunk, 0)


@functools.partial(
    pl.kernel,
    out_type=jax.ShapeDtypeStruct((NC, N_PAD), jnp.float32),
    mesh=_MESH,
    scratch_types=[
        pltpu.VMEM((NB_ALL // (NC * NS), EB), jnp.int32),
        pltpu.VMEM((EB,), jnp.int32),
        pltpu.VMEM((EB,), jnp.float32),
        pltpu.VMEM((N_PAD // NS,), jnp.float32),
        pltpu.VMEM_SHARED((N_PAD,), jnp.float32),
        pltpu.SemaphoreType.DMA,
    ],
)
def _deg_sc(dst2_hbm, cnt_hbm, didx2, dbuf, ones_v, bounce, acc, sem):
    c = lax.axis_index("c")
    s = lax.axis_index("s")
    rpt = N_PAD // NS
    nb = NB_ALL // (NC * NS)

    one16 = jnp.ones((LANES,), jnp.float32)
    zero16 = jnp.zeros((LANES,), jnp.float32)
    for j in range(EB // LANES):
        ones_v[pl.ds(j * LANES, LANES)] = one16

    def _zb(i, carry):
        bounce[pl.ds(i * LANES, LANES)] = zero16
        return carry

    lax.fori_loop(0, rpt // LANES, _zb, 0)
    w = c * NS + s
    pltpu.sync_copy(dst2_hbm.at[pl.ds(w * nb, nb)], didx2)
    pltpu.sync_copy(bounce, acc.at[pl.ds(s * rpt, rpt)])
    plsc.subcore_barrier()

    def _body(b, carry):
        for j in range(EB // LANES):
            dbuf[pl.ds(j * LANES, LANES)] = didx2[b, pl.ds(j * LANES, LANES)]
        pltpu.sync_copy(ones_v, acc.at[dbuf], add=True)
        return carry

    lax.fori_loop(0, nb, _body, 0)
    plsc.subcore_barrier()

    pltpu.sync_copy(acc.at[pl.ds(s * rpt, rpt)], bounce)
    pltpu.sync_copy(bounce, cnt_hbm.at[c, pl.ds(s * rpt, rpt)])


_DH1 = D_HID // 2
_NB1 = E_PAD // (NS * EB)


@functools.partial(
    pl.kernel,
    out_type=(
        jax.ShapeDtypeStruct((N_PAD, _DH1), jnp.float32),
        jax.ShapeDtypeStruct((N_PAD, _DH1), jnp.float32),
    ),
    mesh=_MESH,
    scratch_types=[
        pltpu.VMEM((CHUNK, EB), jnp.int32),
        pltpu.VMEM((CHUNK, EB), jnp.int32),
        pltpu.VMEM((EB,), jnp.int32),
        pltpu.VMEM((EB, _DH1), jnp.float32),
        pltpu.VMEM((EB, _DH1), jnp.float32),
        pltpu.VMEM_SHARED((N_PAD, _DH1), jnp.float32),
        pltpu.SemaphoreType.DMA,
        pltpu.SemaphoreType.DMA,
    ],
)
def _prop_l1(src2_hbm, dst2_hbm, ta_hbm, tb_hbm, outa_hbm, outb_hbm,
             sidx2, didx2, dbuf, rows0, rows1, acc, sem0, sem1):
    c = lax.axis_index("c")
    s = lax.axis_index("s")
    rpt = N_PAD // NS

    _zero_rows(rows0, EB, _DH1)
    base = s * rpt
    for k in range(rpt // EB):
        pltpu.sync_copy(rows0, acc.at[pl.ds(base + k * EB, EB)])
    plsc.subcore_barrier()

    @pl.when(c == 0)
    def _():
        _edge_pipeline(ta_hbm, src2_hbm, dst2_hbm, s * _NB1, _NB1,
                       sidx2, didx2, dbuf, rows0, rows1, sem0, sem1, acc)

    @pl.when(c == 1)
    def _():
        _edge_pipeline(tb_hbm, src2_hbm, dst2_hbm, s * _NB1, _NB1,
                       sidx2, didx2, dbuf, rows0, rows1, sem0, sem1, acc)

    plsc.subcore_barrier()

    def _drain(out_hbm):
        pltpu.sync_copy(acc.at[pl.ds(base, rpt)], out_hbm.at[pl.ds(base, rpt)])

    @pl.when(c == 0)
    def _():
        _drain(outa_hbm)

    @pl.when(c == 1)
    def _():
        _drain(outb_hbm)


_NB2 = E_PAD // (NC * NS * EB)


@functools.partial(
    pl.kernel,
    out_type=(
        jax.ShapeDtypeStruct((N_PAD, D_OUT), jnp.float32),
        jax.ShapeDtypeStruct((N_PAD, D_OUT), jnp.float32),
    ),
    mesh=_MESH,
    scratch_types=[
        pltpu.VMEM((CHUNK, EB), jnp.int32),
        pltpu.VMEM((CHUNK, EB), jnp.int32),
        pltpu.VMEM((EB,), jnp.int32),
        pltpu.VMEM((EB, D_OUT), jnp.float32),
        pltpu.VMEM((EB, D_OUT), jnp.float32),
        pltpu.VMEM_SHARED((N_PAD, D_OUT), jnp.float32),
        pltpu.SemaphoreType.DMA,
        pltpu.SemaphoreType.DMA,
    ],
)
def _prop_l2(src2_hbm, dst2_hbm, t_hbm, outa_hbm, outb_hbm,
             sidx2, didx2, dbuf, rows0, rows1, acc, sem0, sem1):
    c = lax.axis_index("c")
    s = lax.axis_index("s")
    rpt = N_PAD // NS

    _zero_rows(rows0, EB, D_OUT)
    w = c * NS + s
    base = s * rpt
    for k in range(rpt // EB):
        pltpu.sync_copy(rows0, acc.at[pl.ds(base + k * EB, EB)])
    plsc.subcore_barrier()

    _edge_pipeline(t_hbm, src2_hbm, dst2_hbm, w * _NB2, _NB2,
                   sidx2, didx2, dbuf, rows0, rows1, sem0, sem1, acc)

    plsc.subcore_barrier()

    def _drain(out_hbm):
        pltpu.sync_copy(acc.at[pl.ds(base, rpt)], out_hbm.at[pl.ds(base, rpt)])

    @pl.when(c == 0)
    def _():
        _drain(outa_hbm)

    @pl.when(c == 1)
    def _():
        _drain(outb_hbm)


def _mm1_body(x_ref, w_ref, cnt_ref, ta_ref, tb_ref, dinv_ref):
    cnt = cnt_ref[0, :] + cnt_ref[1, :]
    dinv = lax.rsqrt(1.0 + cnt)
    h = jnp.dot(x_ref[...], w_ref[...], preferred_element_type=jnp.float32)
    h = h * dinv[:, None]
    ta_ref[...] = h[:, : D_HID // 2]
    tb_ref[...] = h[:, D_HID // 2 :]
    dinv_ref[...] = dinv


def _mm1(x_p, w1, cnt):
    return pl.pallas_call(
        _mm1_body,
        grid=(GRID_R,),
        in_specs=[
            pl.BlockSpec((ROW_BLK, D_IN), lambda i: (i, 0)),
            pl.BlockSpec((D_IN, D_HID), lambda i: (0, 0)),
            pl.BlockSpec((NC, ROW_BLK), lambda i: (0, i)),
        ],
        compiler_params=pltpu.CompilerParams(
            dimension_semantics=("arbitrary",),
        ),
        out_specs=[
            pl.BlockSpec((ROW_BLK, D_HID // 2), lambda i: (i, 0)),
            pl.BlockSpec((ROW_BLK, D_HID // 2), lambda i: (i, 0)),
            pl.BlockSpec((ROW_BLK,), lambda i: (i,)),
        ],
        out_shape=[
            jax.ShapeDtypeStruct((N_PAD, D_HID // 2), jnp.float32),
            jax.ShapeDtypeStruct((N_PAD, D_HID // 2), jnp.float32),
            jax.ShapeDtypeStruct((N_PAD,), jnp.float32),
        ],
    )(x_p, w1, cnt)


def _mm2_body(aa_ref, ab_ref, ta_ref, tb_ref, dinv_ref, b1_ref, w2_ref,
              t2_ref):
    dinv = dinv_ref[...]
    h1 = jnp.concatenate(
        [aa_ref[...] + ta_ref[...], ab_ref[...] + tb_ref[...]], axis=1
    )
    out1 = jnp.maximum(h1 * dinv[:, None] + b1_ref[...][None, :], 0.0)
    h2 = jnp.dot(out1, w2_ref[...], preferred_element_type=jnp.float32)
    t2_ref[...] = h2 * dinv[:, None]


def _mm2(aa, ab, ta, tb, dinv, b1, w2):
    return pl.pallas_call(
        _mm2_body,
        grid=(GRID_R,),
        in_specs=[
            pl.BlockSpec((ROW_BLK, D_HID // 2), lambda i: (i, 0)),
            pl.BlockSpec((ROW_BLK, D_HID // 2), lambda i: (i, 0)),
            pl.BlockSpec((ROW_BLK, D_HID // 2), lambda i: (i, 0)),
            pl.BlockSpec((ROW_BLK, D_HID // 2), lambda i: (i, 0)),
            pl.BlockSpec((ROW_BLK,), lambda i: (i,)),
            pl.BlockSpec((D_HID,), lambda i: (0,)),
            pl.BlockSpec((D_HID, D_OUT), lambda i: (0, 0)),
        ],
        out_specs=pl.BlockSpec((ROW_BLK, D_OUT), lambda i: (i, 0)),
        out_shape=jax.ShapeDtypeStruct((N_PAD, D_OUT), jnp.float32),
    )(aa, ab, ta, tb, dinv, b1, w2)


def _final_body(aa_ref, ab_ref, t2_ref, dinv_ref, b2_ref, out_ref):
    dinv = dinv_ref[...]
    h = aa_ref[...] + ab_ref[...] + t2_ref[...]
    out_ref[...] = h * dinv[:, None] + b2_ref[...][None, :]


def _final(aa, ab, t2, dinv, b2):
    return pl.pallas_call(
        _final_body,
        grid=(GRID_R,),
        in_specs=[
            pl.BlockSpec((ROW_BLK, D_OUT), lambda i: (i, 0)),
            pl.BlockSpec((ROW_BLK, D_OUT), lambda i: (i, 0)),
            pl.BlockSpec((ROW_BLK, D_OUT), lambda i: (i, 0)),
            pl.BlockSpec((ROW_BLK,), lambda i: (i,)),
            pl.BlockSpec((D_OUT,), lambda i: (0,)),
        ],
        out_specs=pl.BlockSpec((ROW_BLK, D_OUT), lambda i: (i, 0)),
        out_shape=jax.ShapeDtypeStruct((N, D_OUT), jnp.float32),
    )(aa, ab, t2, dinv, b2)


def kernel(x, edge_index, W1, b1, W2, b2):
    src = edge_index[0]
    dst = edge_index[1]
    pad_e = E_PAD - E
    pad_i = jnp.arange(pad_e, dtype=jnp.int32)
    src_p = jnp.concatenate([src, pad_i % N]).reshape(NB_ALL, EB)
    dst_p = jnp.concatenate(
        [dst, N + pad_i % (N_PAD - N)]
    ).reshape(NB_ALL, EB)
    cnt = _deg_sc(dst_p)
    ta, tb, dinv = _mm1(x, W1, cnt)
    aa, ab = _prop_l1(src_p, dst_p, ta, tb)
    t2 = _mm2(aa, ab, ta, tb, dinv, b1, W2)
    a2a, a2b = _prop_l2(src_p, dst_p, t2)
    out = _final(a2a, a2b, t2, dinv, b2)
    return out


# speedup vs baseline: 1.0926x; 1.0658x over previous
import functools

import jax
import jax.numpy as jnp
from jax import lax
from jax.experimental import pallas as pl
from jax.experimental.pallas import tpu as pltpu
from jax.experimental.pallas import tpu_sc as plsc

N = 10000
E = 160000
D_IN = 256
D_HID = 256
D_OUT = 128

NC = 2
NS = 16
LANES = 16

N_PAD = 10240
E_PAD = 163840
EB = 128
NB_ALL = E_PAD // EB
ROW_BLK = 1024
GRID_R = N_PAD // ROW_BLK

_MESH = plsc.VectorSubcoreMesh(
    core_axis_name="c", subcore_axis_name="s", num_cores=NC, num_subcores=NS
)


def _zero_rows(rows, nrow, ncol):
    zero16 = jnp.zeros((LANES,), jnp.float32)

    def _zr(i, carry):
        def _zc(j, carry2):
            rows[i, pl.ds(j * LANES, LANES)] = zero16
            return carry2

        return lax.fori_loop(0, ncol // LANES, _zc, carry)

    lax.fori_loop(0, nrow, _zr, 0)


CHUNK = 40


def _edge_pipeline(t_hbm, src2_hbm, dst2_hbm, batch0, nb,
                   sidx2, didx2, dbuf, rows0, rows1, sem0, sem1, acc):
    def _scatter(b, rows):
        for j in range(EB // LANES):
            dbuf[pl.ds(j * LANES, LANES)] = didx2[b, pl.ds(j * LANES, LANES)]
        pltpu.sync_copy(rows, acc.at[dbuf], add=True)

    def _chunk(ci, carry):
        boff = batch0 + ci * CHUNK
        pltpu.sync_copy(src2_hbm.at[pl.ds(boff, CHUNK)], sidx2)
        pltpu.sync_copy(dst2_hbm.at[pl.ds(boff, CHUNK)], didx2)
        pltpu.async_copy(t_hbm.at[sidx2.at[0]], rows0, sem0)

        def _body(b2, carry2):
            b = b2 * 2
            pltpu.async_copy(t_hbm.at[sidx2.at[b + 1]], rows1, sem1)
            pltpu.make_async_copy(t_hbm.at[sidx2.at[b]], rows0, sem0).wait()
            _scatter(b, rows0)
            pltpu.async_copy(t_hbm.at[sidx2.at[b + 2]], rows0, sem0)
            pltpu.make_async_copy(
                t_hbm.at[sidx2.at[b + 1]], rows1, sem1).wait()
            _scatter(b + 1, rows1)
            return carry2

        lax.fori_loop(0, CHUNK // 2 - 1, _body, 0)
        b = CHUNK - 2
        pltpu.async_copy(t_hbm.at[sidx2.at[b + 1]], rows1, sem1)
        pltpu.make_async_copy(t_hbm.at[sidx2.at[b]], rows0, sem0).wait()
        _scatter(b, rows0)
        pltpu.make_async_copy(t_hbm.at[sidx2.at[b + 1]], rows1, sem1).wait()
        _scatter(b + 1, rows1)
        return carry

    lax.fori_loop(0, nb // CHUNK, _chunk, 0)


@functools.partial(
    pl.kernel,
    out_type=jax.ShapeDtypeStruct((NC, N_PAD), jnp.float32),
    mesh=_MESH,
    scratch_types=[
        pltpu.VMEM((NB_ALL // (NC * NS), EB), jnp.int32),
        pltpu.VMEM((EB,), jnp.int32),
        pltpu.VMEM((EB,), jnp.float32),
        pltpu.VMEM((N_PAD // NS,), jnp.float32),
        pltpu.VMEM_SHARED((N_PAD,), jnp.float32),
        pltpu.SemaphoreType.DMA,
    ],
)
def _deg_sc(dst2_hbm, cnt_hbm, didx2, dbuf, ones_v, bounce, acc, sem):
    c = lax.axis_index("c")
    s = lax.axis_index("s")
    rpt = N_PAD // NS
    nb = NB_ALL // (NC * NS)

    one16 = jnp.ones((LANES,), jnp.float32)
    zero16 = jnp.zeros((LANES,), jnp.float32)
    for j in range(EB // LANES):
        ones_v[pl.ds(j * LANES, LANES)] = one16

    def _zb(i, carry):
        bounce[pl.ds(i * LANES, LANES)] = zero16
        return carry

    lax.fori_loop(0, rpt // LANES, _zb, 0)
    w = c * NS + s
    pltpu.sync_copy(dst2_hbm.at[pl.ds(w * nb, nb)], didx2)
    pltpu.sync_copy(bounce, acc.at[pl.ds(s * rpt, rpt)])
    plsc.subcore_barrier()

    def _body(b, carry):
        for j in range(EB // LANES):
            dbuf[pl.ds(j * LANES, LANES)] = didx2[b, pl.ds(j * LANES, LANES)]
        pltpu.sync_copy(ones_v, acc.at[dbuf], add=True)
        return carry

    lax.fori_loop(0, nb, _body, 0)
    plsc.subcore_barrier()

    pltpu.sync_copy(acc.at[pl.ds(s * rpt, rpt)], bounce)
    pltpu.sync_copy(bounce, cnt_hbm.at[c, pl.ds(s * rpt, rpt)])


_DH1 = D_HID // 2
_NB1 = E_PAD // (NS * EB)


@functools.partial(
    pl.kernel,
    out_type=(
        jax.ShapeDtypeStruct((N_PAD, _DH1), jnp.float32),
        jax.ShapeDtypeStruct((N_PAD, _DH1), jnp.float32),
    ),
    mesh=_MESH,
    scratch_types=[
        pltpu.VMEM((CHUNK, EB), jnp.int32),
        pltpu.VMEM((CHUNK, EB), jnp.int32),
        pltpu.VMEM((EB,), jnp.int32),
        pltpu.VMEM((EB, _DH1), jnp.float32),
        pltpu.VMEM((EB, _DH1), jnp.float32),
        pltpu.VMEM_SHARED((N_PAD, _DH1), jnp.float32),
        pltpu.SemaphoreType.DMA,
        pltpu.SemaphoreType.DMA,
    ],
)
def _prop_l1(src2_hbm, dst2_hbm, ta_hbm, tb_hbm, outa_hbm, outb_hbm,
             sidx2, didx2, dbuf, rows0, rows1, acc, sem0, sem1):
    c = lax.axis_index("c")
    s = lax.axis_index("s")
    rpt = N_PAD // NS

    _zero_rows(rows0, EB, _DH1)
    base = s * rpt
    for k in range(rpt // EB):
        pltpu.sync_copy(rows0, acc.at[pl.ds(base + k * EB, EB)])
    plsc.subcore_barrier()

    @pl.when(c == 0)
    def _():
        _edge_pipeline(ta_hbm, src2_hbm, dst2_hbm, s * _NB1, _NB1,
                       sidx2, didx2, dbuf, rows0, rows1, sem0, sem1, acc)

    @pl.when(c == 1)
    def _():
        _edge_pipeline(tb_hbm, src2_hbm, dst2_hbm, s * _NB1, _NB1,
                       sidx2, didx2, dbuf, rows0, rows1, sem0, sem1, acc)

    plsc.subcore_barrier()

    def _drain(out_hbm):
        pltpu.sync_copy(acc.at[pl.ds(base, rpt)], out_hbm.at[pl.ds(base, rpt)])

    @pl.when(c == 0)
    def _():
        _drain(outa_hbm)

    @pl.when(c == 1)
    def _():
        _drain(outb_hbm)


_NB2 = E_PAD // (NC * NS * EB)


@functools.partial(
    pl.kernel,
    out_type=(
        jax.ShapeDtypeStruct((N_PAD, D_OUT), jnp.float32),
        jax.ShapeDtypeStruct((N_PAD, D_OUT), jnp.float32),
    ),
    mesh=_MESH,
    scratch_types=[
        pltpu.VMEM((CHUNK, EB), jnp.int32),
        pltpu.VMEM((CHUNK, EB), jnp.int32),
        pltpu.VMEM((EB,), jnp.int32),
        pltpu.VMEM((EB, D_OUT), jnp.float32),
        pltpu.VMEM((EB, D_OUT), jnp.float32),
        pltpu.VMEM_SHARED((N_PAD, D_OUT), jnp.float32),
        pltpu.SemaphoreType.DMA,
        pltpu.SemaphoreType.DMA,
    ],
)
def _prop_l2(src2_hbm, dst2_hbm, t_hbm, outa_hbm, outb_hbm,
             sidx2, didx2, dbuf, rows0, rows1, acc, sem0, sem1):
    c = lax.axis_index("c")
    s = lax.axis_index("s")
    rpt = N_PAD // NS

    _zero_rows(rows0, EB, D_OUT)
    w = c * NS + s
    base = s * rpt
    for k in range(rpt // EB):
        pltpu.sync_copy(rows0, acc.at[pl.ds(base + k * EB, EB)])
    plsc.subcore_barrier()

    _edge_pipeline(t_hbm, src2_hbm, dst2_hbm, w * _NB2, _NB2,
                   sidx2, didx2, dbuf, rows0, rows1, sem0, sem1, acc)

    plsc.subcore_barrier()

    def _drain(out_hbm):
        pltpu.sync_copy(acc.at[pl.ds(base, rpt)], out_hbm.at[pl.ds(base, rpt)])

    @pl.when(c == 0)
    def _():
        _drain(outa_hbm)

    @pl.when(c == 1)
    def _():
        _drain(outb_hbm)


def _mm1_body(x_ref, w_ref, cnt_ref, ta_ref, tb_ref, dinv_ref):
    cnt = cnt_ref[0, :] + cnt_ref[1, :]
    dinv = lax.rsqrt(1.0 + cnt)
    h = jnp.dot(x_ref[...], w_ref[...], preferred_element_type=jnp.float32)
    h = h * dinv[:, None]
    ta_ref[...] = h[:, : D_HID // 2]
    tb_ref[...] = h[:, D_HID // 2 :]
    dinv_ref[...] = dinv


def _mm1(x_p, w1, cnt):
    return pl.pallas_call(
        _mm1_body,
        grid=(GRID_R,),
        in_specs=[
            pl.BlockSpec((ROW_BLK, D_IN), lambda i: (i, 0)),
            pl.BlockSpec((D_IN, D_HID), lambda i: (0, 0)),
            pl.BlockSpec((NC, ROW_BLK), lambda i: (0, i)),
        ],
        compiler_params=pltpu.CompilerParams(
            dimension_semantics=("arbitrary",),
        ),
        out_specs=[
            pl.BlockSpec((ROW_BLK, D_HID // 2), lambda i: (i, 0)),
            pl.BlockSpec((ROW_BLK, D_HID // 2), lambda i: (i, 0)),
            pl.BlockSpec((ROW_BLK,), lambda i: (i,)),
        ],
        out_shape=[
            jax.ShapeDtypeStruct((N_PAD, D_HID // 2), jnp.float32),
            jax.ShapeDtypeStruct((N_PAD, D_HID // 2), jnp.float32),
            jax.ShapeDtypeStruct((N_PAD,), jnp.float32),
        ],
    )(x_p, w1, cnt)


def _mm2_body(aa_ref, ab_ref, ta_ref, tb_ref, dinv_ref, b1_ref, w2_ref,
              t2_ref):
    dinv = dinv_ref[...]
    h1 = jnp.concatenate(
        [aa_ref[...] + ta_ref[...], ab_ref[...] + tb_ref[...]], axis=1
    )
    out1 = jnp.maximum(h1 * dinv[:, None] + b1_ref[...][None, :], 0.0)
    h2 = jnp.dot(out1, w2_ref[...], preferred_element_type=jnp.float32)
    t2_ref[...] = h2 * dinv[:, None]


def _mm2(aa, ab, ta, tb, dinv, b1, w2):
    return pl.pallas_call(
        _mm2_body,
        grid=(GRID_R,),
        in_specs=[
            pl.BlockSpec((ROW_BLK, D_HID // 2), lambda i: (i, 0)),
            pl.BlockSpec((ROW_BLK, D_HID // 2), lambda i: (i, 0)),
            pl.BlockSpec((ROW_BLK, D_HID // 2), lambda i: (i, 0)),
            pl.BlockSpec((ROW_BLK, D_HID // 2), lambda i: (i, 0)),
            pl.BlockSpec((ROW_BLK,), lambda i: (i,)),
            pl.BlockSpec((D_HID,), lambda i: (0,)),
            pl.BlockSpec((D_HID, D_OUT), lambda i: (0, 0)),
        ],
        out_specs=pl.BlockSpec((ROW_BLK, D_OUT), lambda i: (i, 0)),
        out_shape=jax.ShapeDtypeStruct((N_PAD, D_OUT), jnp.float32),
    )(aa, ab, ta, tb, dinv, b1, w2)


def _final_body(aa_ref, ab_ref, t2_ref, dinv_ref, b2_ref, out_ref):
    dinv = dinv_ref[...]
    h = aa_ref[...] + ab_ref[...] + t2_ref[...]
    out_ref[...] = h * dinv[:, None] + b2_ref[...][None, :]


def _final(aa, ab, t2, dinv, b2):
    return pl.pallas_call(
        _final_body,
        grid=(GRID_R,),
        in_specs=[
            pl.BlockSpec((ROW_BLK, D_OUT), lambda i: (i, 0)),
            pl.BlockSpec((ROW_BLK, D_OUT), lambda i: (i, 0)),
            pl.BlockSpec((ROW_BLK, D_OUT), lambda i: (i, 0)),
            pl.BlockSpec((ROW_BLK,), lambda i: (i,)),
            pl.BlockSpec((D_OUT,), lambda i: (0,)),
        ],
        out_specs=pl.BlockSpec((ROW_BLK, D_OUT), lambda i: (i, 0)),
        out_shape=jax.ShapeDtypeStruct((N, D_OUT), jnp.float32),
    )(aa, ab, t2, dinv, b2)


def kernel(x, edge_index, W1, b1, W2, b2):
    src = edge_index[0]
    dst = edge_index[1]
    pad_e = E_PAD - E
    pad_i = jnp.arange(pad_e, dtype=jnp.int32)
    src_p = jnp.concatenate([src, pad_i % N]).reshape(NB_ALL, EB)
    dst_p = jnp.concatenate(
        [dst, N + pad_i % (N_PAD - N)]
    ).reshape(NB_ALL, EB)
    cnt = _deg_sc(dst_p)
    ta, tb, dinv = _mm1(x, W1, cnt)
    aa, ab = _prop_l1(src_p, dst_p, ta, tb)
    t2 = _mm2(aa, ab, ta, tb, dinv, b1, W2)
    a2a, a2b = _prop_l2(src_p, dst_p, t2)
    out = _final(a2a, a2b, t2, dinv, b2)
    return out


# speedup vs baseline: 1.1190x; 1.0241x over previous
import functools

import jax
import jax.numpy as jnp
from jax import lax
from jax.experimental import pallas as pl
from jax.experimental.pallas import tpu as pltpu
from jax.experimental.pallas import tpu_sc as plsc

N = 10000
E = 160000
D_IN = 256
D_HID = 256
D_OUT = 128

NC = 2
NS = 16
LANES = 16

N_PAD = 10240
E_PAD = 163840
EB = 128
NB_ALL = E_PAD // EB
ROW_BLK = 2048
GRID_R = N_PAD // ROW_BLK

_MESH = plsc.VectorSubcoreMesh(
    core_axis_name="c", subcore_axis_name="s", num_cores=NC, num_subcores=NS
)


def _zero_rows(rows, nrow, ncol):
    zero16 = jnp.zeros((LANES,), jnp.float32)

    def _zr(i, carry):
        def _zc(j, carry2):
            rows[i, pl.ds(j * LANES, LANES)] = zero16
            return carry2

        return lax.fori_loop(0, ncol // LANES, _zc, carry)

    lax.fori_loop(0, nrow, _zr, 0)


CHUNK = 40


def _edge_pipeline(t_hbm, src2_hbm, dst2_hbm, batch0, nb,
                   sidx2, didx2, dbuf, rows0, rows1, sem0, sem1, acc):
    def _scatter(b, rows):
        for j in range(EB // LANES):
            dbuf[pl.ds(j * LANES, LANES)] = didx2[b, pl.ds(j * LANES, LANES)]
        pltpu.sync_copy(rows, acc.at[dbuf], add=True)

    def _chunk(ci, carry):
        boff = batch0 + ci * CHUNK
        pltpu.sync_copy(src2_hbm.at[pl.ds(boff, CHUNK)], sidx2)
        pltpu.sync_copy(dst2_hbm.at[pl.ds(boff, CHUNK)], didx2)
        pltpu.async_copy(t_hbm.at[sidx2.at[0]], rows0, sem0)

        def _body(b2, carry2):
            b = b2 * 2
            pltpu.async_copy(t_hbm.at[sidx2.at[b + 1]], rows1, sem1)
            pltpu.make_async_copy(t_hbm.at[sidx2.at[b]], rows0, sem0).wait()
            _scatter(b, rows0)
            pltpu.async_copy(t_hbm.at[sidx2.at[b + 2]], rows0, sem0)
            pltpu.make_async_copy(
                t_hbm.at[sidx2.at[b + 1]], rows1, sem1).wait()
            _scatter(b + 1, rows1)
            return carry2

        lax.fori_loop(0, CHUNK // 2 - 1, _body, 0)
        b = CHUNK - 2
        pltpu.async_copy(t_hbm.at[sidx2.at[b + 1]], rows1, sem1)
        pltpu.make_async_copy(t_hbm.at[sidx2.at[b]], rows0, sem0).wait()
        _scatter(b, rows0)
        pltpu.make_async_copy(t_hbm.at[sidx2.at[b + 1]], rows1, sem1).wait()
        _scatter(b + 1, rows1)
        return carry

    lax.fori_loop(0, nb // CHUNK, _chunk, 0)


@functools.partial(
    pl.kernel,
    out_type=jax.ShapeDtypeStruct((NC, N_PAD), jnp.float32),
    mesh=_MESH,
    scratch_types=[
        pltpu.VMEM((NB_ALL // (NC * NS), EB), jnp.int32),
        pltpu.VMEM((EB,), jnp.int32),
        pltpu.VMEM((EB,), jnp.float32),
        pltpu.VMEM((N_PAD // NS,), jnp.float32),
        pltpu.VMEM_SHARED((N_PAD,), jnp.float32),
        pltpu.SemaphoreType.DMA,
    ],
)
def _deg_sc(dst2_hbm, cnt_hbm, didx2, dbuf, ones_v, bounce, acc, sem):
    c = lax.axis_index("c")
    s = lax.axis_index("s")
    rpt = N_PAD // NS
    nb = NB_ALL // (NC * NS)

    one16 = jnp.ones((LANES,), jnp.float32)
    zero16 = jnp.zeros((LANES,), jnp.float32)
    for j in range(EB // LANES):
        ones_v[pl.ds(j * LANES, LANES)] = one16

    def _zb(i, carry):
        bounce[pl.ds(i * LANES, LANES)] = zero16
        return carry

    lax.fori_loop(0, rpt // LANES, _zb, 0)
    w = c * NS + s
    pltpu.sync_copy(dst2_hbm.at[pl.ds(w * nb, nb)], didx2)
    pltpu.sync_copy(bounce, acc.at[pl.ds(s * rpt, rpt)])
    plsc.subcore_barrier()

    def _body(b, carry):
        for j in range(EB // LANES):
            dbuf[pl.ds(j * LANES, LANES)] = didx2[b, pl.ds(j * LANES, LANES)]
        pltpu.sync_copy(ones_v, acc.at[dbuf], add=True)
        return carry

    lax.fori_loop(0, nb, _body, 0)
    plsc.subcore_barrier()

    pltpu.sync_copy(acc.at[pl.ds(s * rpt, rpt)], bounce)
    pltpu.sync_copy(bounce, cnt_hbm.at[c, pl.ds(s * rpt, rpt)])


_DH1 = D_HID // 2
_NB1 = E_PAD // (NS * EB)


@functools.partial(
    pl.kernel,
    out_type=(
        jax.ShapeDtypeStruct((N_PAD, _DH1), jnp.float32),
        jax.ShapeDtypeStruct((N_PAD, _DH1), jnp.float32),
    ),
    mesh=_MESH,
    scratch_types=[
        pltpu.VMEM((CHUNK, EB), jnp.int32),
        pltpu.VMEM((CHUNK, EB), jnp.int32),
        pltpu.VMEM((EB,), jnp.int32),
        pltpu.VMEM((EB, _DH1), jnp.float32),
        pltpu.VMEM((EB, _DH1), jnp.float32),
        pltpu.VMEM_SHARED((N_PAD, _DH1), jnp.float32),
        pltpu.SemaphoreType.DMA,
        pltpu.SemaphoreType.DMA,
    ],
)
def _prop_l1(src2_hbm, dst2_hbm, ta_hbm, tb_hbm, outa_hbm, outb_hbm,
             sidx2, didx2, dbuf, rows0, rows1, acc, sem0, sem1):
    c = lax.axis_index("c")
    s = lax.axis_index("s")
    rpt = N_PAD // NS

    _zero_rows(rows0, EB, _DH1)
    base = s * rpt
    for k in range(rpt // EB):
        pltpu.sync_copy(rows0, acc.at[pl.ds(base + k * EB, EB)])
    plsc.subcore_barrier()

    @pl.when(c == 0)
    def _():
        _edge_pipeline(ta_hbm, src2_hbm, dst2_hbm, s * _NB1, _NB1,
                       sidx2, didx2, dbuf, rows0, rows1, sem0, sem1, acc)

    @pl.when(c == 1)
    def _():
        _edge_pipeline(tb_hbm, src2_hbm, dst2_hbm, s * _NB1, _NB1,
                       sidx2, didx2, dbuf, rows0, rows1, sem0, sem1, acc)

    plsc.subcore_barrier()

    def _drain(out_hbm):
        pltpu.sync_copy(acc.at[pl.ds(base, rpt)], out_hbm.at[pl.ds(base, rpt)])

    @pl.when(c == 0)
    def _():
        _drain(outa_hbm)

    @pl.when(c == 1)
    def _():
        _drain(outb_hbm)


_NB2 = E_PAD // (NC * NS * EB)


@functools.partial(
    pl.kernel,
    out_type=(
        jax.ShapeDtypeStruct((N_PAD, D_OUT), jnp.float32),
        jax.ShapeDtypeStruct((N_PAD, D_OUT), jnp.float32),
    ),
    mesh=_MESH,
    scratch_types=[
        pltpu.VMEM((CHUNK, EB), jnp.int32),
        pltpu.VMEM((CHUNK, EB), jnp.int32),
        pltpu.VMEM((EB,), jnp.int32),
        pltpu.VMEM((EB, D_OUT), jnp.float32),
        pltpu.VMEM((EB, D_OUT), jnp.float32),
        pltpu.VMEM_SHARED((N_PAD, D_OUT), jnp.float32),
        pltpu.SemaphoreType.DMA,
        pltpu.SemaphoreType.DMA,
    ],
)
def _prop_l2(src2_hbm, dst2_hbm, t_hbm, outa_hbm, outb_hbm,
             sidx2, didx2, dbuf, rows0, rows1, acc, sem0, sem1):
    c = lax.axis_index("c")
    s = lax.axis_index("s")
    rpt = N_PAD // NS

    _zero_rows(rows0, EB, D_OUT)
    w = c * NS + s
    base = s * rpt
    for k in range(rpt // EB):
        pltpu.sync_copy(rows0, acc.at[pl.ds(base + k * EB, EB)])
    plsc.subcore_barrier()

    _edge_pipeline(t_hbm, src2_hbm, dst2_hbm, w * _NB2, _NB2,
                   sidx2, didx2, dbuf, rows0, rows1, sem0, sem1, acc)

    plsc.subcore_barrier()

    def _drain(out_hbm):
        pltpu.sync_copy(acc.at[pl.ds(base, rpt)], out_hbm.at[pl.ds(base, rpt)])

    @pl.when(c == 0)
    def _():
        _drain(outa_hbm)

    @pl.when(c == 1)
    def _():
        _drain(outb_hbm)


def _mm1_body(x_ref, w_ref, cnt_ref, ta_ref, tb_ref, dinv_ref):
    cnt = cnt_ref[0, :] + cnt_ref[1, :]
    dinv = lax.rsqrt(1.0 + cnt)
    h = jnp.dot(x_ref[...], w_ref[...], preferred_element_type=jnp.float32)
    h = h * dinv[:, None]
    ta_ref[...] = h[:, : D_HID // 2]
    tb_ref[...] = h[:, D_HID // 2 :]
    dinv_ref[...] = dinv


def _mm1(x_p, w1, cnt):
    return pl.pallas_call(
        _mm1_body,
        grid=(GRID_R,),
        in_specs=[
            pl.BlockSpec((ROW_BLK, D_IN), lambda i: (i, 0)),
            pl.BlockSpec((D_IN, D_HID), lambda i: (0, 0)),
            pl.BlockSpec((NC, ROW_BLK), lambda i: (0, i)),
        ],
        compiler_params=pltpu.CompilerParams(
            dimension_semantics=("arbitrary",),
        ),
        out_specs=[
            pl.BlockSpec((ROW_BLK, D_HID // 2), lambda i: (i, 0)),
            pl.BlockSpec((ROW_BLK, D_HID // 2), lambda i: (i, 0)),
            pl.BlockSpec((ROW_BLK,), lambda i: (i,)),
        ],
        out_shape=[
            jax.ShapeDtypeStruct((N_PAD, D_HID // 2), jnp.float32),
            jax.ShapeDtypeStruct((N_PAD, D_HID // 2), jnp.float32),
            jax.ShapeDtypeStruct((N_PAD,), jnp.float32),
        ],
    )(x_p, w1, cnt)


def _mm2_body(aa_ref, ab_ref, ta_ref, tb_ref, dinv_ref, b1_ref, w2_ref,
              t2_ref):
    dinv = dinv_ref[...]
    h1 = jnp.concatenate(
        [aa_ref[...] + ta_ref[...], ab_ref[...] + tb_ref[...]], axis=1
    )
    out1 = jnp.maximum(h1 * dinv[:, None] + b1_ref[...][None, :], 0.0)
    h2 = jnp.dot(out1, w2_ref[...], preferred_element_type=jnp.float32)
    t2_ref[...] = h2 * dinv[:, None]


def _mm2(aa, ab, ta, tb, dinv, b1, w2):
    return pl.pallas_call(
        _mm2_body,
        grid=(GRID_R,),
        in_specs=[
            pl.BlockSpec((ROW_BLK, D_HID // 2), lambda i: (i, 0)),
            pl.BlockSpec((ROW_BLK, D_HID // 2), lambda i: (i, 0)),
            pl.BlockSpec((ROW_BLK, D_HID // 2), lambda i: (i, 0)),
            pl.BlockSpec((ROW_BLK, D_HID // 2), lambda i: (i, 0)),
            pl.BlockSpec((ROW_BLK,), lambda i: (i,)),
            pl.BlockSpec((D_HID,), lambda i: (0,)),
            pl.BlockSpec((D_HID, D_OUT), lambda i: (0, 0)),
        ],
        out_specs=pl.BlockSpec((ROW_BLK, D_OUT), lambda i: (i, 0)),
        out_shape=jax.ShapeDtypeStruct((N_PAD, D_OUT), jnp.float32),
    )(aa, ab, ta, tb, dinv, b1, w2)


def _final_body(aa_ref, ab_ref, t2_ref, dinv_ref, b2_ref, out_ref):
    dinv = dinv_ref[...]
    h = aa_ref[...] + ab_ref[...] + t2_ref[...]
    out_ref[...] = h * dinv[:, None] + b2_ref[...][None, :]


def _final(aa, ab, t2, dinv, b2):
    return pl.pallas_call(
        _final_body,
        grid=(GRID_R,),
        in_specs=[
            pl.BlockSpec((ROW_BLK, D_OUT), lambda i: (i, 0)),
            pl.BlockSpec((ROW_BLK, D_OUT), lambda i: (i, 0)),
            pl.BlockSpec((ROW_BLK, D_OUT), lambda i: (i, 0)),
            pl.BlockSpec((ROW_BLK,), lambda i: (i,)),
            pl.BlockSpec((D_OUT,), lambda i: (0,)),
        ],
        out_specs=pl.BlockSpec((ROW_BLK, D_OUT), lambda i: (i, 0)),
        out_shape=jax.ShapeDtypeStruct((N, D_OUT), jnp.float32),
    )(aa, ab, t2, dinv, b2)


def kernel(x, edge_index, W1, b1, W2, b2):
    src = edge_index[0]
    dst = edge_index[1]
    pad_e = E_PAD - E
    pad_i = jnp.arange(pad_e, dtype=jnp.int32)
    src_p = jnp.concatenate([src, pad_i % N]).reshape(NB_ALL, EB)
    dst_p = jnp.concatenate(
        [dst, N + pad_i % (N_PAD - N)]
    ).reshape(NB_ALL, EB)
    cnt = _deg_sc(dst_p)
    ta, tb, dinv = _mm1(x, W1, cnt)
    aa, ab = _prop_l1(src_p, dst_p, ta, tb)
    t2 = _mm2(aa, ab, ta, tb, dinv, b1, W2)
    a2a, a2b = _prop_l2(src_p, dst_p, t2)
    out = _final(a2a, a2b, t2, dinv, b2)
    return out


# speedup vs baseline: 1.1442x; 1.0225x over previous
import functools

import jax
import jax.numpy as jnp
from jax import lax
from jax.experimental import pallas as pl
from jax.experimental.pallas import tpu as pltpu
from jax.experimental.pallas import tpu_sc as plsc

N = 10000
E = 160000
D_IN = 256
D_HID = 256
D_OUT = 128

NC = 2
NS = 16
LANES = 16

N_PAD = 10240
E_PAD = 163840
EB = 128
NB_ALL = E_PAD // EB
ROW_BLK = 5120
GRID_R = N_PAD // ROW_BLK

_MESH = plsc.VectorSubcoreMesh(
    core_axis_name="c", subcore_axis_name="s", num_cores=NC, num_subcores=NS
)


def _zero_rows(rows, nrow, ncol):
    zero16 = jnp.zeros((LANES,), jnp.float32)

    def _zr(i, carry):
        def _zc(j, carry2):
            rows[i, pl.ds(j * LANES, LANES)] = zero16
            return carry2

        return lax.fori_loop(0, ncol // LANES, _zc, carry)

    lax.fori_loop(0, nrow, _zr, 0)


CHUNK = 40


def _edge_pipeline(t_hbm, src2_hbm, dst2_hbm, batch0, nb,
                   sidx2, didx2, dbuf, rows0, rows1, sem0, sem1, acc):
    def _scatter(b, rows):
        for j in range(EB // LANES):
            dbuf[pl.ds(j * LANES, LANES)] = didx2[b, pl.ds(j * LANES, LANES)]
        pltpu.sync_copy(rows, acc.at[dbuf], add=True)

    def _chunk(ci, carry):
        boff = batch0 + ci * CHUNK
        pltpu.sync_copy(src2_hbm.at[pl.ds(boff, CHUNK)], sidx2)
        pltpu.sync_copy(dst2_hbm.at[pl.ds(boff, CHUNK)], didx2)
        pltpu.async_copy(t_hbm.at[sidx2.at[0]], rows0, sem0)

        def _body(b2, carry2):
            b = b2 * 2
            pltpu.async_copy(t_hbm.at[sidx2.at[b + 1]], rows1, sem1)
            pltpu.make_async_copy(t_hbm.at[sidx2.at[b]], rows0, sem0).wait()
            _scatter(b, rows0)
            pltpu.async_copy(t_hbm.at[sidx2.at[b + 2]], rows0, sem0)
            pltpu.make_async_copy(
                t_hbm.at[sidx2.at[b + 1]], rows1, sem1).wait()
            _scatter(b + 1, rows1)
            return carry2

        lax.fori_loop(0, CHUNK // 2 - 1, _body, 0)
        b = CHUNK - 2
        pltpu.async_copy(t_hbm.at[sidx2.at[b + 1]], rows1, sem1)
        pltpu.make_async_copy(t_hbm.at[sidx2.at[b]], rows0, sem0).wait()
        _scatter(b, rows0)
        pltpu.make_async_copy(t_hbm.at[sidx2.at[b + 1]], rows1, sem1).wait()
        _scatter(b + 1, rows1)
        return carry

    lax.fori_loop(0, nb // CHUNK, _chunk, 0)


@functools.partial(
    pl.kernel,
    out_type=jax.ShapeDtypeStruct((NC, N_PAD), jnp.float32),
    mesh=_MESH,
    scratch_types=[
        pltpu.VMEM((NB_ALL // (NC * NS), EB), jnp.int32),
        pltpu.VMEM((EB,), jnp.int32),
        pltpu.VMEM((EB,), jnp.float32),
        pltpu.VMEM((N_PAD // NS,), jnp.float32),
        pltpu.VMEM_SHARED((N_PAD,), jnp.float32),
        pltpu.SemaphoreType.DMA,
    ],
)
def _deg_sc(dst2_hbm, cnt_hbm, didx2, dbuf, ones_v, bounce, acc, sem):
    c = lax.axis_index("c")
    s = lax.axis_index("s")
    rpt = N_PAD // NS
    nb = NB_ALL // (NC * NS)

    one16 = jnp.ones((LANES,), jnp.float32)
    zero16 = jnp.zeros((LANES,), jnp.float32)
    for j in range(EB // LANES):
        ones_v[pl.ds(j * LANES, LANES)] = one16

    def _zb(i, carry):
        bounce[pl.ds(i * LANES, LANES)] = zero16
        return carry

    lax.fori_loop(0, rpt // LANES, _zb, 0)
    w = c * NS + s
    pltpu.sync_copy(dst2_hbm.at[pl.ds(w * nb, nb)], didx2)
    pltpu.sync_copy(bounce, acc.at[pl.ds(s * rpt, rpt)])
    plsc.subcore_barrier()

    def _body(b, carry):
        for j in range(EB // LANES):
            dbuf[pl.ds(j * LANES, LANES)] = didx2[b, pl.ds(j * LANES, LANES)]
        pltpu.sync_copy(ones_v, acc.at[dbuf], add=True)
        return carry

    lax.fori_loop(0, nb, _body, 0)
    plsc.subcore_barrier()

    pltpu.sync_copy(acc.at[pl.ds(s * rpt, rpt)], bounce)
    pltpu.sync_copy(bounce, cnt_hbm.at[c, pl.ds(s * rpt, rpt)])


_DH1 = D_HID // 2
_NB1 = E_PAD // (NS * EB)


@functools.partial(
    pl.kernel,
    out_type=(
        jax.ShapeDtypeStruct((N_PAD, _DH1), jnp.float32),
        jax.ShapeDtypeStruct((N_PAD, _DH1), jnp.float32),
    ),
    mesh=_MESH,
    scratch_types=[
        pltpu.VMEM((CHUNK, EB), jnp.int32),
        pltpu.VMEM((CHUNK, EB), jnp.int32),
        pltpu.VMEM((EB,), jnp.int32),
        pltpu.VMEM((EB, _DH1), jnp.float32),
        pltpu.VMEM((EB, _DH1), jnp.float32),
        pltpu.VMEM_SHARED((N_PAD, _DH1), jnp.float32),
        pltpu.SemaphoreType.DMA,
        pltpu.SemaphoreType.DMA,
    ],
)
def _prop_l1(src2_hbm, dst2_hbm, ta_hbm, tb_hbm, outa_hbm, outb_hbm,
             sidx2, didx2, dbuf, rows0, rows1, acc, sem0, sem1):
    c = lax.axis_index("c")
    s = lax.axis_index("s")
    rpt = N_PAD // NS

    _zero_rows(rows0, EB, _DH1)
    base = s * rpt
    for k in range(rpt // EB):
        pltpu.sync_copy(rows0, acc.at[pl.ds(base + k * EB, EB)])
    plsc.subcore_barrier()

    @pl.when(c == 0)
    def _():
        _edge_pipeline(ta_hbm, src2_hbm, dst2_hbm, s * _NB1, _NB1,
                       sidx2, didx2, dbuf, rows0, rows1, sem0, sem1, acc)

    @pl.when(c == 1)
    def _():
        _edge_pipeline(tb_hbm, src2_hbm, dst2_hbm, s * _NB1, _NB1,
                       sidx2, didx2, dbuf, rows0, rows1, sem0, sem1, acc)

    plsc.subcore_barrier()

    def _drain(out_hbm):
        pltpu.sync_copy(acc.at[pl.ds(base, rpt)], out_hbm.at[pl.ds(base, rpt)])

    @pl.when(c == 0)
    def _():
        _drain(outa_hbm)

    @pl.when(c == 1)
    def _():
        _drain(outb_hbm)


_NB2 = E_PAD // (NC * NS * EB)


@functools.partial(
    pl.kernel,
    out_type=(
        jax.ShapeDtypeStruct((N_PAD, D_OUT), jnp.float32),
        jax.ShapeDtypeStruct((N_PAD, D_OUT), jnp.float32),
    ),
    mesh=_MESH,
    scratch_types=[
        pltpu.VMEM((CHUNK, EB), jnp.int32),
        pltpu.VMEM((CHUNK, EB), jnp.int32),
        pltpu.VMEM((EB,), jnp.int32),
        pltpu.VMEM((EB, D_OUT), jnp.float32),
        pltpu.VMEM((EB, D_OUT), jnp.float32),
        pltpu.VMEM_SHARED((N_PAD, D_OUT), jnp.float32),
        pltpu.SemaphoreType.DMA,
        pltpu.SemaphoreType.DMA,
    ],
)
def _prop_l2(src2_hbm, dst2_hbm, t_hbm, outa_hbm, outb_hbm,
             sidx2, didx2, dbuf, rows0, rows1, acc, sem0, sem1):
    c = lax.axis_index("c")
    s = lax.axis_index("s")
    rpt = N_PAD // NS

    _zero_rows(rows0, EB, D_OUT)
    w = c * NS + s
    base = s * rpt
    for k in range(rpt // EB):
        pltpu.sync_copy(rows0, acc.at[pl.ds(base + k * EB, EB)])
    plsc.subcore_barrier()

    _edge_pipeline(t_hbm, src2_hbm, dst2_hbm, w * _NB2, _NB2,
                   sidx2, didx2, dbuf, rows0, rows1, sem0, sem1, acc)

    plsc.subcore_barrier()

    def _drain(out_hbm):
        pltpu.sync_copy(acc.at[pl.ds(base, rpt)], out_hbm.at[pl.ds(base, rpt)])

    @pl.when(c == 0)
    def _():
        _drain(outa_hbm)

    @pl.when(c == 1)
    def _():
        _drain(outb_hbm)


def _mm1_body(x_ref, w_ref, cnt_ref, ta_ref, tb_ref, dinv_ref):
    cnt = cnt_ref[0, :] + cnt_ref[1, :]
    dinv = lax.rsqrt(1.0 + cnt)
    h = jnp.dot(x_ref[...], w_ref[...], preferred_element_type=jnp.float32)
    h = h * dinv[:, None]
    ta_ref[...] = h[:, : D_HID // 2]
    tb_ref[...] = h[:, D_HID // 2 :]
    dinv_ref[...] = dinv


def _mm1(x_p, w1, cnt):
    return pl.pallas_call(
        _mm1_body,
        grid=(GRID_R,),
        in_specs=[
            pl.BlockSpec((ROW_BLK, D_IN), lambda i: (i, 0)),
            pl.BlockSpec((D_IN, D_HID), lambda i: (0, 0)),
            pl.BlockSpec((NC, ROW_BLK), lambda i: (0, i)),
        ],
        compiler_params=pltpu.CompilerParams(
            dimension_semantics=("arbitrary",),
        ),
        out_specs=[
            pl.BlockSpec((ROW_BLK, D_HID // 2), lambda i: (i, 0)),
            pl.BlockSpec((ROW_BLK, D_HID // 2), lambda i: (i, 0)),
            pl.BlockSpec((ROW_BLK,), lambda i: (i,)),
        ],
        out_shape=[
            jax.ShapeDtypeStruct((N_PAD, D_HID // 2), jnp.float32),
            jax.ShapeDtypeStruct((N_PAD, D_HID // 2), jnp.float32),
            jax.ShapeDtypeStruct((N_PAD,), jnp.float32),
        ],
    )(x_p, w1, cnt)


def _mm2_body(aa_ref, ab_ref, ta_ref, tb_ref, dinv_ref, b1_ref, w2_ref,
              t2_ref):
    dinv = dinv_ref[...]
    h1 = jnp.concatenate(
        [aa_ref[...] + ta_ref[...], ab_ref[...] + tb_ref[...]], axis=1
    )
    out1 = jnp.maximum(h1 * dinv[:, None] + b1_ref[...][None, :], 0.0)
    h2 = jnp.dot(out1, w2_ref[...], preferred_element_type=jnp.float32)
    t2_ref[...] = h2 * dinv[:, None]


def _mm2(aa, ab, ta, tb, dinv, b1, w2):
    return pl.pallas_call(
        _mm2_body,
        grid=(GRID_R,),
        in_specs=[
            pl.BlockSpec((ROW_BLK, D_HID // 2), lambda i: (i, 0)),
            pl.BlockSpec((ROW_BLK, D_HID // 2), lambda i: (i, 0)),
            pl.BlockSpec((ROW_BLK, D_HID // 2), lambda i: (i, 0)),
            pl.BlockSpec((ROW_BLK, D_HID // 2), lambda i: (i, 0)),
            pl.BlockSpec((ROW_BLK,), lambda i: (i,)),
            pl.BlockSpec((D_HID,), lambda i: (0,)),
            pl.BlockSpec((D_HID, D_OUT), lambda i: (0, 0)),
        ],
        out_specs=pl.BlockSpec((ROW_BLK, D_OUT), lambda i: (i, 0)),
        out_shape=jax.ShapeDtypeStruct((N_PAD, D_OUT), jnp.float32),
    )(aa, ab, ta, tb, dinv, b1, w2)


def _final_body(aa_ref, ab_ref, t2_ref, dinv_ref, b2_ref, out_ref):
    dinv = dinv_ref[...]
    h = aa_ref[...] + ab_ref[...] + t2_ref[...]
    out_ref[...] = h * dinv[:, None] + b2_ref[...][None, :]


def _final(aa, ab, t2, dinv, b2):
    return pl.pallas_call(
        _final_body,
        grid=(GRID_R,),
        in_specs=[
            pl.BlockSpec((ROW_BLK, D_OUT), lambda i: (i, 0)),
            pl.BlockSpec((ROW_BLK, D_OUT), lambda i: (i, 0)),
            pl.BlockSpec((ROW_BLK, D_OUT), lambda i: (i, 0)),
            pl.BlockSpec((ROW_BLK,), lambda i: (i,)),
            pl.BlockSpec((D_OUT,), lambda i: (0,)),
        ],
        out_specs=pl.BlockSpec((ROW_BLK, D_OUT), lambda i: (i, 0)),
        out_shape=jax.ShapeDtypeStruct((N, D_OUT), jnp.float32),
    )(aa, ab, t2, dinv, b2)


def kernel(x, edge_index, W1, b1, W2, b2):
    src = edge_index[0]
    dst = edge_index[1]
    pad_e = E_PAD - E
    pad_i = jnp.arange(pad_e, dtype=jnp.int32)
    src_p = jnp.concatenate([src, pad_i % N]).reshape(NB_ALL, EB)
    dst_p = jnp.concatenate(
        [dst, N + pad_i % (N_PAD - N)]
    ).reshape(NB_ALL, EB)
    cnt = _deg_sc(dst_p)
    ta, tb, dinv = _mm1(x, W1, cnt)
    aa, ab = _prop_l1(src_p, dst_p, ta, tb)
    t2 = _mm2(aa, ab, ta, tb, dinv, b1, W2)
    a2a, a2b = _prop_l2(src_p, dst_p, t2)
    out = _final(a2a, a2b, t2, dinv, b2)
    return out


# speedup vs baseline: 1.1632x; 1.0166x over previous
import functools

import jax
import jax.numpy as jnp
from jax import lax
from jax.experimental import pallas as pl
from jax.experimental.pallas import tpu as pltpu
from jax.experimental.pallas import tpu_sc as plsc

N = 10000
E = 160000
D_IN = 256
D_HID = 256
D_OUT = 128

NC = 2
NS = 16
LANES = 16

N_PAD = 10240
E_PAD = 163840
EB = 128
NB_ALL = E_PAD // EB
ROW_BLK = 5120
GRID_R = N_PAD // ROW_BLK

_MESH = plsc.VectorSubcoreMesh(
    core_axis_name="c", subcore_axis_name="s", num_cores=NC, num_subcores=NS
)


def _zero_rows(rows, nrow, ncol):
    zero16 = jnp.zeros((LANES,), jnp.float32)

    def _zr(i, carry):
        def _zc(j, carry2):
            rows[i, pl.ds(j * LANES, LANES)] = zero16
            return carry2

        return lax.fori_loop(0, ncol // LANES, _zc, carry)

    lax.fori_loop(0, nrow, _zr, 0)


CHUNK = 40


def _stage_idx(src2_hbm, dst2_hbm, boff, sidx2, didx2):
    pltpu.sync_copy(src2_hbm.at[pl.ds(boff, CHUNK)], sidx2)
    pltpu.sync_copy(dst2_hbm.at[pl.ds(boff, CHUNK)], didx2)


def _edge_pipeline(t_hbm, src2_hbm, dst2_hbm, batch0, nb,
                   sidx2, didx2, dbuf, rows0, rows1, sem0, sem1, acc):
    def _scatter(b, rows):
        for j in range(EB // LANES):
            dbuf[pl.ds(j * LANES, LANES)] = didx2[b, pl.ds(j * LANES, LANES)]
        pltpu.sync_copy(rows, acc.at[dbuf], add=True)

    def _run_chunk():
        pltpu.async_copy(t_hbm.at[sidx2.at[0]], rows0, sem0)

        def _body(b2, carry2):
            b = b2 * 2
            pltpu.async_copy(t_hbm.at[sidx2.at[b + 1]], rows1, sem1)
            pltpu.make_async_copy(t_hbm.at[sidx2.at[b]], rows0, sem0).wait()
            _scatter(b, rows0)
            pltpu.async_copy(t_hbm.at[sidx2.at[b + 2]], rows0, sem0)
            pltpu.make_async_copy(
                t_hbm.at[sidx2.at[b + 1]], rows1, sem1).wait()
            _scatter(b + 1, rows1)
            return carry2

        lax.fori_loop(0, CHUNK // 2 - 1, _body, 0)
        b = CHUNK - 2
        pltpu.async_copy(t_hbm.at[sidx2.at[b + 1]], rows1, sem1)
        pltpu.make_async_copy(t_hbm.at[sidx2.at[b]], rows0, sem0).wait()
        _scatter(b, rows0)
        pltpu.make_async_copy(t_hbm.at[sidx2.at[b + 1]], rows1, sem1).wait()
        _scatter(b + 1, rows1)

    _run_chunk()
    for ci in range(1, nb // CHUNK):
        _stage_idx(src2_hbm, dst2_hbm, batch0 + ci * CHUNK, sidx2, didx2)
        _run_chunk()


@functools.partial(
    pl.kernel,
    out_type=jax.ShapeDtypeStruct((NC, N_PAD), jnp.float32),
    mesh=_MESH,
    scratch_types=[
        pltpu.VMEM((NB_ALL // (NC * NS), EB), jnp.int32),
        pltpu.VMEM((EB,), jnp.int32),
        pltpu.VMEM((EB,), jnp.float32),
        pltpu.VMEM((N_PAD // NS,), jnp.float32),
        pltpu.VMEM_SHARED((N_PAD,), jnp.float32),
        pltpu.SemaphoreType.DMA,
    ],
)
def _deg_sc(dst2_hbm, cnt_hbm, didx2, dbuf, ones_v, bounce, acc, sem):
    c = lax.axis_index("c")
    s = lax.axis_index("s")
    rpt = N_PAD // NS
    nb = NB_ALL // (NC * NS)

    one16 = jnp.ones((LANES,), jnp.float32)
    zero16 = jnp.zeros((LANES,), jnp.float32)
    for j in range(EB // LANES):
        ones_v[pl.ds(j * LANES, LANES)] = one16

    def _zb(i, carry):
        bounce[pl.ds(i * LANES, LANES)] = zero16
        return carry

    lax.fori_loop(0, rpt // LANES, _zb, 0)
    w = c * NS + s
    pltpu.sync_copy(dst2_hbm.at[pl.ds(w * nb, nb)], didx2)
    pltpu.sync_copy(bounce, acc.at[pl.ds(s * rpt, rpt)])
    plsc.subcore_barrier()

    def _body(b, carry):
        for j in range(EB // LANES):
            dbuf[pl.ds(j * LANES, LANES)] = didx2[b, pl.ds(j * LANES, LANES)]
        pltpu.sync_copy(ones_v, acc.at[dbuf], add=True)
        return carry

    lax.fori_loop(0, nb, _body, 0)
    plsc.subcore_barrier()

    pltpu.sync_copy(acc.at[pl.ds(s * rpt, rpt)], bounce)
    pltpu.sync_copy(bounce, cnt_hbm.at[c, pl.ds(s * rpt, rpt)])


_DH1 = D_HID // 2
_NB1 = E_PAD // (NS * EB)


@functools.partial(
    pl.kernel,
    out_type=(
        jax.ShapeDtypeStruct((N_PAD, _DH1), jnp.float32),
        jax.ShapeDtypeStruct((N_PAD, _DH1), jnp.float32),
    ),
    mesh=_MESH,
    scratch_types=[
        pltpu.VMEM((CHUNK, EB), jnp.int32),
        pltpu.VMEM((CHUNK, EB), jnp.int32),
        pltpu.VMEM((EB,), jnp.int32),
        pltpu.VMEM((EB, _DH1), jnp.float32),
        pltpu.VMEM((EB, _DH1), jnp.float32),
        pltpu.VMEM_SHARED((N_PAD, _DH1), jnp.float32),
        pltpu.SemaphoreType.DMA,
        pltpu.SemaphoreType.DMA,
    ],
)
def _prop_l1(src2_hbm, dst2_hbm, ta_hbm, tb_hbm, outa_hbm, outb_hbm,
             sidx2, didx2, dbuf, rows0, rows1, acc, sem0, sem1):
    c = lax.axis_index("c")
    s = lax.axis_index("s")
    rpt = N_PAD // NS

    _zero_rows(rows0, EB, _DH1)
    base = s * rpt
    zd = [
        pltpu.async_copy(rows0, acc.at[pl.ds(base + k * EB, EB)], sem1)
        for k in range(rpt // EB)
    ]
    _stage_idx(src2_hbm, dst2_hbm, s * _NB1, sidx2, didx2)
    for d in zd:
        d.wait()
    plsc.subcore_barrier()

    @pl.when(c == 0)
    def _():
        _edge_pipeline(ta_hbm, src2_hbm, dst2_hbm, s * _NB1, _NB1,
                       sidx2, didx2, dbuf, rows0, rows1, sem0, sem1, acc)

    @pl.when(c == 1)
    def _():
        _edge_pipeline(tb_hbm, src2_hbm, dst2_hbm, s * _NB1, _NB1,
                       sidx2, didx2, dbuf, rows0, rows1, sem0, sem1, acc)

    plsc.subcore_barrier()

    def _drain(out_hbm):
        pltpu.sync_copy(acc.at[pl.ds(base, rpt)], out_hbm.at[pl.ds(base, rpt)])

    @pl.when(c == 0)
    def _():
        _drain(outa_hbm)

    @pl.when(c == 1)
    def _():
        _drain(outb_hbm)


_NB2 = E_PAD // (NC * NS * EB)


@functools.partial(
    pl.kernel,
    out_type=(
        jax.ShapeDtypeStruct((N_PAD, D_OUT), jnp.float32),
        jax.ShapeDtypeStruct((N_PAD, D_OUT), jnp.float32),
    ),
    mesh=_MESH,
    scratch_types=[
        pltpu.VMEM((CHUNK, EB), jnp.int32),
        pltpu.VMEM((CHUNK, EB), jnp.int32),
        pltpu.VMEM((EB,), jnp.int32),
        pltpu.VMEM((EB, D_OUT), jnp.float32),
        pltpu.VMEM((EB, D_OUT), jnp.float32),
        pltpu.VMEM_SHARED((N_PAD, D_OUT), jnp.float32),
        pltpu.SemaphoreType.DMA,
        pltpu.SemaphoreType.DMA,
    ],
)
def _prop_l2(src2_hbm, dst2_hbm, t_hbm, outa_hbm, outb_hbm,
             sidx2, didx2, dbuf, rows0, rows1, acc, sem0, sem1):
    c = lax.axis_index("c")
    s = lax.axis_index("s")
    rpt = N_PAD // NS

    _zero_rows(rows0, EB, D_OUT)
    w = c * NS + s
    base = s * rpt
    zd = [
        pltpu.async_copy(rows0, acc.at[pl.ds(base + k * EB, EB)], sem1)
        for k in range(rpt // EB)
    ]
    _stage_idx(src2_hbm, dst2_hbm, w * _NB2, sidx2, didx2)
    for d in zd:
        d.wait()
    plsc.subcore_barrier()

    _edge_pipeline(t_hbm, src2_hbm, dst2_hbm, w * _NB2, _NB2,
                   sidx2, didx2, dbuf, rows0, rows1, sem0, sem1, acc)

    plsc.subcore_barrier()

    def _drain(out_hbm):
        pltpu.sync_copy(acc.at[pl.ds(base, rpt)], out_hbm.at[pl.ds(base, rpt)])

    @pl.when(c == 0)
    def _():
        _drain(outa_hbm)

    @pl.when(c == 1)
    def _():
        _drain(outb_hbm)


def _mm1_body(x_ref, w_ref, cnt_ref, ta_ref, tb_ref, dinv_ref):
    cnt = cnt_ref[0, :] + cnt_ref[1, :]
    dinv = lax.rsqrt(1.0 + cnt)
    h = jnp.dot(x_ref[...], w_ref[...], preferred_element_type=jnp.float32)
    h = h * dinv[:, None]
    ta_ref[...] = h[:, : D_HID // 2]
    tb_ref[...] = h[:, D_HID // 2 :]
    dinv_ref[...] = dinv


def _mm1(x_p, w1, cnt):
    return pl.pallas_call(
        _mm1_body,
        grid=(GRID_R,),
        in_specs=[
            pl.BlockSpec((ROW_BLK, D_IN), lambda i: (i, 0)),
            pl.BlockSpec((D_IN, D_HID), lambda i: (0, 0)),
            pl.BlockSpec((NC, ROW_BLK), lambda i: (0, i)),
        ],
        compiler_params=pltpu.CompilerParams(
            dimension_semantics=("arbitrary",),
        ),
        out_specs=[
            pl.BlockSpec((ROW_BLK, D_HID // 2), lambda i: (i, 0)),
            pl.BlockSpec((ROW_BLK, D_HID // 2), lambda i: (i, 0)),
            pl.BlockSpec((ROW_BLK,), lambda i: (i,)),
        ],
        out_shape=[
            jax.ShapeDtypeStruct((N_PAD, D_HID // 2), jnp.float32),
            jax.ShapeDtypeStruct((N_PAD, D_HID // 2), jnp.float32),
            jax.ShapeDtypeStruct((N_PAD,), jnp.float32),
        ],
    )(x_p, w1, cnt)


def _mm2_body(aa_ref, ab_ref, ta_ref, tb_ref, dinv_ref, b1_ref, w2_ref,
              t2_ref):
    dinv = dinv_ref[...]
    h1 = jnp.concatenate(
        [aa_ref[...] + ta_ref[...], ab_ref[...] + tb_ref[...]], axis=1
    )
    out1 = jnp.maximum(h1 * dinv[:, None] + b1_ref[...][None, :], 0.0)
    h2 = jnp.dot(out1, w2_ref[...], preferred_element_type=jnp.float32)
    t2_ref[...] = h2 * dinv[:, None]


def _mm2(aa, ab, ta, tb, dinv, b1, w2):
    return pl.pallas_call(
        _mm2_body,
        grid=(GRID_R,),
        in_specs=[
            pl.BlockSpec((ROW_BLK, D_HID // 2), lambda i: (i, 0)),
            pl.BlockSpec((ROW_BLK, D_HID // 2), lambda i: (i, 0)),
            pl.BlockSpec((ROW_BLK, D_HID // 2), lambda i: (i, 0)),
            pl.BlockSpec((ROW_BLK, D_HID // 2), lambda i: (i, 0)),
            pl.BlockSpec((ROW_BLK,), lambda i: (i,)),
            pl.BlockSpec((D_HID,), lambda i: (0,)),
            pl.BlockSpec((D_HID, D_OUT), lambda i: (0, 0)),
        ],
        out_specs=pl.BlockSpec((ROW_BLK, D_OUT), lambda i: (i, 0)),
        out_shape=jax.ShapeDtypeStruct((N_PAD, D_OUT), jnp.float32),
    )(aa, ab, ta, tb, dinv, b1, w2)


def _final_body(aa_ref, ab_ref, t2_ref, dinv_ref, b2_ref, out_ref):
    dinv = dinv_ref[...]
    h = aa_ref[...] + ab_ref[...] + t2_ref[...]
    out_ref[...] = h * dinv[:, None] + b2_ref[...][None, :]


def _final(aa, ab, t2, dinv, b2):
    return pl.pallas_call(
        _final_body,
        grid=(GRID_R,),
        in_specs=[
            pl.BlockSpec((ROW_BLK, D_OUT), lambda i: (i, 0)),
            pl.BlockSpec((ROW_BLK, D_OUT), lambda i: (i, 0)),
            pl.BlockSpec((ROW_BLK, D_OUT), lambda i: (i, 0)),
            pl.BlockSpec((ROW_BLK,), lambda i: (i,)),
            pl.BlockSpec((D_OUT,), lambda i: (0,)),
        ],
        out_specs=pl.BlockSpec((ROW_BLK, D_OUT), lambda i: (i, 0)),
        out_shape=jax.ShapeDtypeStruct((N, D_OUT), jnp.float32),
    )(aa, ab, t2, dinv, b2)


def kernel(x, edge_index, W1, b1, W2, b2):
    src = edge_index[0]
    dst = edge_index[1]
    pad_e = E_PAD - E
    pad_i = jnp.arange(pad_e, dtype=jnp.int32)
    src_p = jnp.concatenate([src, pad_i % N]).reshape(NB_ALL, EB)
    dst_p = jnp.concatenate(
        [dst, N + pad_i % (N_PAD - N)]
    ).reshape(NB_ALL, EB)
    cnt = _deg_sc(dst_p)
    ta, tb, dinv = _mm1(x, W1, cnt)
    aa, ab = _prop_l1(src_p, dst_p, ta, tb)
    t2 = _mm2(aa, ab, ta, tb, dinv, b1, W2)
    a2a, a2b = _prop_l2(src_p, dst_p, t2)
    out = _final(a2a, a2b, t2, dinv, b2)
    return out
